# baseline (speedup 1.0000x reference)
import functools

import jax
import jax.numpy as jnp
from jax import lax
from jax.experimental import pallas as pl
from jax.experimental.pallas import tpu as pltpu
from jax.experimental.pallas import tpu_sc as plsc

_EPS = 1e-6
_CH = 16
_NW = 32
_L = 16


def _uload(u_v, k):
    return u_v[k // 8, pl.ds((k % 8) * _L, _L)]


def _rsqrt16(v):
    vv = jnp.maximum(v, jnp.float32(1e-30))
    i = lax.bitcast_convert_type(vv, jnp.int32)
    i = jnp.int32(0x5F3759DF) - lax.shift_right_logical(i, 1)
    r = lax.bitcast_convert_type(i, jnp.float32)
    for _ in range(3):
        r = r * (jnp.float32(1.5) - jnp.float32(0.5) * vv * r * r)
    return r


def _make_sc_call(T, C, D, dtype):
    n_atoms = T // D
    n_chunks = n_atoms // _CH
    rows = _CH * D
    max_chunks_per_w = -(-n_chunks // _NW)
    mesh = plsc.VectorSubcoreMesh(core_axis_name="c", subcore_axis_name="s")

    @functools.partial(
        pl.kernel,
        mesh=mesh,
        out_type=jax.ShapeDtypeStruct((T, C), dtype),
        scratch_types=[
            pltpu.VMEM((rows, C), jnp.float32),
            pltpu.VMEM((D * D // 8, 128), jnp.float32),
        ],
    )
    def sc_call(x_hbm, u_hbm, out_hbm, y_v, u_v):
        wid = lax.axis_index("s") * 2 + lax.axis_index("c")
        pltpu.sync_copy(u_hbm, u_v)

        def do_chunk(ci, _):
            c = wid + ci * _NW

            @pl.when(c < n_chunks)
            def _():
                row0 = c * rows
                pltpu.sync_copy(x_hbm.at[pl.ds(row0, rows)], y_v)

                def do_atom(a, _):
                    base = a * D

                    def do_cc(cc, _):
                        col = cc * _L
                        ys = [y_v[base + j, pl.ds(col, _L)] for j in range(D)]
                        parts = [jnp.zeros((_L,), jnp.float32) for _ in range(4)]
                        for i in range(D):
                            prods = [_uload(u_v, i * D + j) * ys[j]
                                     for j in range(i, D)]
                            while len(prods) > 1:
                                nxt = [prods[k] + prods[k + 1]
                                       for k in range(0, len(prods) - 1, 2)]
                                if len(prods) % 2:
                                    nxt.append(prods[-1])
                                prods = nxt
                            parts[i % 4] = parts[i % 4] + ys[i] * prods[0]
                        norm = (parts[0] + parts[1]) + (parts[2] + parts[3])
                        r = _rsqrt16(norm)
                        inv = jnp.float32(1.0) / (norm * r + jnp.float32(_EPS))
                        for i in range(D):
                            y_v[base + i, pl.ds(col, _L)] = ys[i] * inv
                        return 0

                    lax.fori_loop(0, C // _L, do_cc, 0, unroll=False)
                    return 0

                lax.fori_loop(0, _CH, do_atom, 0, unroll=False)
                pltpu.sync_copy(y_v, out_hbm.at[pl.ds(row0, rows)])

            return 0

        lax.fori_loop(0, max_chunks_per_w, do_chunk, 0, unroll=False)

    return sc_call


def kernel(x, atom_mask, S):
    T, C = x.shape
    D = S.shape[0]
    u = jnp.triu(S + S.T) - jnp.diag(jnp.diagonal(S))
    sc_call = _make_sc_call(T, C, D, x.dtype)
    u_splat = jnp.broadcast_to(u.astype(jnp.float32).reshape(-1, 1), (D * D, _L))
    return sc_call(x, u_splat.reshape(D * D // 8, 128))


# speedup vs baseline: 20.0287x; 20.0287x over previous
import jax
import jax.numpy as jnp
from jax.experimental import pallas as pl

_EPS = 1e-6
_A = 8
_K = 50


def _body(x_ref, bd_ref, o_ref):
    R = bd_ref.shape[0]
    C = x_ref.shape[1]
    A = _A
    D = R // A
    bd = bd_ref[:]
    for k in range(_K):
        w = x_ref[pl.ds(k * R, R), :]
        z = jnp.dot(bd, w, preferred_element_type=jnp.float32)
        p = (z * w).reshape(A, D, C)
        norm = jnp.sum(p, axis=1, keepdims=True)
        inv = 1.0 / (jnp.sqrt(norm) + _EPS)
        scale = jnp.broadcast_to(inv, (A, D, C)).reshape(R, C)
        o_ref[pl.ds(k * R, R), :] = w * scale


def kernel(x, atom_mask, S):
    T, C = x.shape
    D = S.shape[0]
    n_atoms = T // D
    A = _A
    R = A * D
    grid = n_atoms // (A * _K)

    bd = jnp.kron(jnp.eye(A, dtype=S.dtype), S)

    out = pl.pallas_call(
        _body,
        grid=(grid,),
        in_specs=[
            pl.BlockSpec((_K * R, C), lambda i: (i, 0)),
            pl.BlockSpec((R, R), lambda i: (0, 0)),
        ],
        out_specs=pl.BlockSpec((_K * R, C), lambda i: (i, 0)),
        out_shape=jax.ShapeDtypeStruct((T, C), x.dtype),
    )(x, bd)
    return out
